# Initial kernel scaffold; baseline (speedup 1.0000x reference)
import functools

import jax
import jax.numpy as jnp
from jax import lax
from jax.experimental import pallas as pl
from jax.experimental.pallas import tpu as pltpu
from jax.experimental.pallas import tpu_sc as plsc

NC = 2
NS = 16
LANES = 16
B = 128


def _sc_spmm(x, src2d, dst2d, w2d):
    n, d = x.shape
    r = src2d.shape[0]
    nrows_sub = n // NS
    zc = 125
    mesh = plsc.VectorSubcoreMesh(core_axis_name="c", subcore_axis_name="s")

    @functools.partial(
        pl.kernel,
        out_type=jax.ShapeDtypeStruct((NC, n, d), jnp.float32),
        mesh=mesh,
        scratch_types=[
            pltpu.VMEM((B,), jnp.int32),
            pltpu.VMEM((B,), jnp.int32),
            pltpu.VMEM((B,), jnp.float32),
            pltpu.VMEM((B, d), jnp.float32),
            pltpu.VMEM_SHARED((n, d), jnp.float32),
            pltpu.SemaphoreType.DMA,
        ],
    )
    def k(x_hbm, src_hbm, dst_hbm, w_hbm, out_hbm, src_v, dst_v, w_v,
          rows_v, acc_sh, sem):
        c = lax.axis_index("c")
        s = lax.axis_index("s")
        wid = s * NC + c

        zero = jnp.zeros((LANES,), jnp.float32)

        @pl.loop(0, zc)
        def _(i):
            for j in range(d // LANES):
                rows_v[i, pl.ds(j * LANES, LANES)] = zero

        base_row = s * nrows_sub
        for t in range(nrows_sub // zc):
            pltpu.sync_copy(rows_v.at[pl.ds(0, zc)],
                            acc_sh.at[pl.ds(base_row + t * zc, zc)])
        plsc.subcore_barrier()

        lo = wid * r // (NC * NS)
        hi = (wid + 1) * r // (NC * NS)

        @pl.loop(lo, hi)
        def _(rr):
            pltpu.sync_copy(src_hbm.at[rr], src_v)
            pltpu.sync_copy(dst_hbm.at[rr], dst_v)
            pltpu.sync_copy(w_hbm.at[rr], w_v)
            pltpu.async_copy(x_hbm.at[src_v], rows_v, sem).wait()

            @pl.loop(0, B)
            def _(i):
                wb = plsc.load_gather(
                    w_v, [jnp.full((LANES,), i, jnp.int32)])
                for j in range(d // LANES):
                    sl = pl.ds(j * LANES, LANES)
                    rows_v[i, sl] = rows_v[i, sl] * wb

            pltpu.sync_copy(rows_v, acc_sh.at[dst_v], add=True)

        plsc.subcore_barrier()

        for t in range(nrows_sub // zc):
            row0 = base_row + t * zc
            pltpu.sync_copy(acc_sh.at[pl.ds(row0, zc)],
                            out_hbm.at[c].at[pl.ds(row0, zc)])

    return k(x, src2d, dst2d, w2d)


def _tc_finish(y, w):
    _, n, d = y.shape
    blk = 2000

    def body(y_ref, w_ref, o_ref):
        acc = y_ref[0] + y_ref[1]
        o_ref[...] = jnp.maximum(
            jnp.dot(acc, w_ref[...], preferred_element_type=jnp.float32), 0.0)

    return pl.pallas_call(
        body,
        out_shape=jax.ShapeDtypeStruct((n, d), jnp.float32),
        grid=(n // blk,),
        in_specs=[
            pl.BlockSpec((2, blk, d), lambda i: (0, i, 0)),
            pl.BlockSpec((d, d), lambda i: (0, 0)),
        ],
        out_specs=pl.BlockSpec((blk, d), lambda i: (i, 0)),
    )(y, W := w)


def kernel(x, edge_index, edge_weight, W):
    e = edge_index.shape[1]
    r = e // B
    src2d = edge_index[0].reshape(r, B)
    dst2d = edge_index[1].reshape(r, B)
    w2d = edge_weight.reshape(r, B)
    y = _sc_spmm(x, src2d, dst2d, w2d)
    return _tc_finish(y, W)


# speedup vs baseline: 4.9232x; 4.9232x over previous
import dataclasses
import functools

import jax
import jax.numpy as jnp
from jax import lax
from jax.experimental import pallas as pl
from jax.experimental.pallas import tpu as pltpu
from jax.experimental.pallas import tpu_sc as plsc

NC = 2
NS = 16
LANES = 16
B = 128
ZC = 80


def _sc_spmm(x, src, dst, w):
    n, d = x.shape
    e = src.shape[0]
    nbatch = e // B
    nchunk = n // ZC
    mesh = plsc.VectorSubcoreMesh(core_axis_name="c", subcore_axis_name="s")
    cp = pltpu.CompilerParams()
    if "needs_layout_passes" in pltpu.CompilerParams.__dataclass_fields__:
        cp = dataclasses.replace(cp, needs_layout_passes=False)

    @functools.partial(
        pl.kernel,
        out_type=jax.ShapeDtypeStruct((NC, n, d), jnp.float32),
        mesh=mesh,
        compiler_params=cp,
        scratch_types=[
            pltpu.VMEM((B,), jnp.int32),
            pltpu.VMEM((B,), jnp.int32),
            pltpu.VMEM((B,), jnp.float32),
            pltpu.VMEM((B, d), jnp.float32),
            pltpu.VMEM_SHARED((n, d), jnp.float32),
            pltpu.SemaphoreType.DMA,
        ],
    )
    def k(x_hbm, src_hbm, dst_hbm, w_hbm, out_hbm, src_v, dst_v, w_v,
          rows_v, acc_sh, sem):
        c = lax.axis_index("c")
        s = lax.axis_index("s")
        wid = s * NC + c

        zero = jnp.zeros((LANES,), jnp.float32)

        @pl.loop(0, ZC)
        def _(i):
            for j in range(d // LANES):
                rows_v[i, pl.ds(j * LANES, LANES)] = zero

        @pl.loop(s, nchunk, step=NS)
        def _(t):
            pltpu.sync_copy(rows_v.at[pl.ds(0, ZC)],
                            acc_sh.at[pl.ds(t * ZC, ZC)])
        plsc.subcore_barrier()

        lo = wid * nbatch // (NC * NS)
        hi = (wid + 1) * nbatch // (NC * NS)

        @pl.loop(lo, hi)
        def _(rr):
            base = rr * B
            pltpu.sync_copy(src_hbm.at[pl.ds(base, B)], src_v)
            pltpu.sync_copy(dst_hbm.at[pl.ds(base, B)], dst_v)
            pltpu.sync_copy(w_hbm.at[pl.ds(base, B)], w_v)
            pltpu.async_copy(x_hbm.at[src_v], rows_v, sem).wait()

            @pl.loop(0, B)
            def _(i):
                wb = plsc.load_gather(
                    w_v, [jnp.full((LANES,), i, jnp.int32)])
                for j in range(d // LANES):
                    sl = pl.ds(j * LANES, LANES)
                    rows_v[i, sl] = rows_v[i, sl] * wb

            pltpu.sync_copy(rows_v, acc_sh.at[dst_v], add=True)

        plsc.subcore_barrier()

        @pl.loop(s, nchunk, step=NS)
        def _(t):
            pltpu.sync_copy(acc_sh.at[pl.ds(t * ZC, ZC)],
                            out_hbm.at[c].at[pl.ds(t * ZC, ZC)])

    return k(x, src, dst, w)


def _tc_finish(y, w):
    _, n, d = y.shape
    blk = 2000

    def body(y_ref, w_ref, o_ref):
        acc = y_ref[0] + y_ref[1]
        o_ref[...] = jnp.maximum(
            jnp.dot(acc, w_ref[...], preferred_element_type=jnp.float32), 0.0)

    return pl.pallas_call(
        body,
        out_shape=jax.ShapeDtypeStruct((n, d), jnp.float32),
        grid=(n // blk,),
        in_specs=[
            pl.BlockSpec((2, blk, d), lambda i: (0, i, 0)),
            pl.BlockSpec((d, d), lambda i: (0, 0)),
        ],
        out_specs=pl.BlockSpec((blk, d), lambda i: (i, 0)),
    )(y, w)


def kernel(x, edge_index, edge_weight, W):
    y = _sc_spmm(x, edge_index[0], edge_index[1], edge_weight)
    return _tc_finish(y, W)
